# baseline (speedup 1.0000x reference)
import functools

import jax
import jax.numpy as jnp
from jax import lax
from jax.experimental import pallas as pl
from jax.experimental.pallas import tpu as pltpu
from jax.experimental.pallas import tpu_sc as plsc

N = 10000
E = 320000
G = 16
H = 128

NC = 2
NS = 16
NW = NC * NS
N_PAD = 10240
EC = 128
CH = 80
SEG = 40
CH0 = 40
CH1 = 2 * CH - CH0
E_PAD = NW * CH * EC
EPT = CH * EC
ROWS_PER_TILE = N_PAD // NS

_mesh = plsc.VectorSubcoreMesh(core_axis_name="c", subcore_axis_name="s")


DW = H


def _deg_body(dst_hbm, zeros_hbm, ones_hbm, deg_out, dst_v, ones_v, deg_sh):
    c = lax.axis_index("c")
    s = lax.axis_index("s")
    w = c * NS + s
    seg = N_PAD // NS

    pltpu.sync_copy(ones_hbm, ones_v)
    pltpu.sync_copy(zeros_hbm, deg_sh.at[pl.ds(s * seg, seg)])
    plsc.subcore_barrier()

    pltpu.sync_copy(dst_hbm.at[pl.ds(w * CH, CH)], dst_v)

    def _chunk(j, _):
        pltpu.sync_copy(ones_v, deg_sh.at[dst_v.at[j]], add=True)
        return 0
    lax.fori_loop(0, CH, _chunk, 0)

    plsc.subcore_barrier()
    sl = pl.ds(s * seg, seg)
    pltpu.sync_copy(deg_sh.at[sl], deg_out.at[c, sl])


_deg_call = pl.kernel(
    _deg_body,
    out_type=jax.ShapeDtypeStruct((NC, N_PAD, DW), jnp.float32),
    mesh=_mesh,
    scratch_types=[
        pltpu.VMEM((CH, EC), jnp.int32),
        pltpu.VMEM((EC, DW), jnp.float32),
        pltpu.VMEM_SHARED((N_PAD, DW), jnp.float32),
    ],
)


def _agg_body(hs_hbm, src_hbm, dst_hbm, zeros_hbm, out_hbm,
              src_v, dst_v, rows_v, acc_sh, sem0, sem1):
    c = lax.axis_index("c")
    s = lax.axis_index("s")
    w = c * NS + s

    pltpu.sync_copy(zeros_hbm, acc_sh.at[pl.ds(s * ROWS_PER_TILE, ROWS_PER_TILE)])
    plsc.subcore_barrier()

    def _core_loop(ch_c, base0):
        def _seg(g, _):
            base = base0 + g * SEG
            pltpu.sync_copy(src_hbm.at[pl.ds(base, SEG)], src_v)
            pltpu.sync_copy(dst_hbm.at[pl.ds(base, SEG)], dst_v)
            pltpu.async_copy(hs_hbm.at[src_v.at[0]], rows_v.at[0], sem0)

            def _pair(p, _):
                pltpu.async_copy(hs_hbm.at[src_v.at[2 * p + 1]], rows_v.at[1], sem1)
                pltpu.make_async_copy(hs_hbm.at[src_v.at[0]], rows_v.at[0], sem0).wait()
                pltpu.sync_copy(rows_v.at[0], acc_sh.at[dst_v.at[2 * p]], add=True)

                @pl.when(p < SEG // 2 - 1)
                def _():
                    pltpu.async_copy(hs_hbm.at[src_v.at[2 * p + 2]], rows_v.at[0], sem0)
                pltpu.make_async_copy(hs_hbm.at[src_v.at[0]], rows_v.at[1], sem1).wait()
                pltpu.sync_copy(rows_v.at[1], acc_sh.at[dst_v.at[2 * p + 1]], add=True)
                return 0
            lax.fori_loop(0, SEG // 2, _pair, 0)
            return 0
        lax.fori_loop(0, ch_c // SEG, _seg, 0)

    @pl.when(c == 0)
    def _():
        _core_loop(CH0, s * CH0)

    @pl.when(c == 1)
    def _():
        _core_loop(CH1, NS * CH0 + s * CH1)

    plsc.subcore_barrier()
    sl = pl.ds(s * ROWS_PER_TILE, ROWS_PER_TILE)
    pltpu.sync_copy(acc_sh.at[sl], out_hbm.at[c, sl])


_agg_call = pl.kernel(
    _agg_body,
    out_type=jax.ShapeDtypeStruct((NC, N_PAD, H), jnp.float32),
    mesh=_mesh,
    scratch_types=[
        pltpu.VMEM((SEG, EC), jnp.int32),
        pltpu.VMEM((SEG, EC), jnp.int32),
        pltpu.VMEM((2, EC, H), jnp.float32),
        pltpu.VMEM_SHARED((N_PAD, H), jnp.float32),
        pltpu.SemaphoreType.DMA,
        pltpu.SemaphoreType.DMA,
    ],
)


def _prep_body(d0_ref, d1_ref, dinv_ref):
    dinv_ref[...] = lax.rsqrt(d0_ref[...] + d1_ref[...] + 1.0)


def _t1_body(y_ref, d_ref, w_ref, o_ref):
    o_ref[...] = jnp.dot(y_ref[...] * d_ref[...], w_ref[...],
                         preferred_element_type=jnp.float32)


BLK = 2000
NBLK = N // BLK


def _t2_body(a0_ref, a1_ref, hs_ref, d_ref, b_ref, bt_ref,
             y_ref, pool_ref, cnt_ref):
    y = jnp.maximum(
        d_ref[...] * (a0_ref[...] + a1_ref[...] + hs_ref[...]) + b_ref[...],
        0.0)
    y_ref[...] = y
    bvec = bt_ref[0, 0, :]
    gi = lax.broadcasted_iota(jnp.int32, (G, BLK), 0)
    oh = (gi == bvec[None, :]).astype(jnp.float32)
    ps = jnp.dot(oh, y, preferred_element_type=jnp.float32)
    cs = jnp.broadcast_to(jnp.sum(oh, axis=1, keepdims=True), (G, H))

    @pl.when(pl.program_id(0) == 0)
    def _():
        pool_ref[...] = jnp.zeros_like(pool_ref)
        cnt_ref[...] = jnp.zeros_like(cnt_ref)

    pool_ref[...] += ps
    cnt_ref[...] += cs


def _head_body(s1_ref, s2_ref, s3_ref, cnt_ref,
               w1_ref, b1_ref, w2_ref, b2_ref, w3_ref, b3_ref, o_ref):
    z = (s1_ref[...] + s2_ref[...] + s3_ref[...]) / jnp.maximum(cnt_ref[...], 1.0)
    z = jnp.maximum(jnp.dot(z, w1_ref[...], preferred_element_type=jnp.float32)
                    + b1_ref[...], 0.0)
    z = jnp.maximum(jnp.dot(z, w2_ref[...], preferred_element_type=jnp.float32)
                    + b2_ref[...], 0.0)
    o_ref[...] = jnp.dot(z, w3_ref[...], preferred_element_type=jnp.float32) \
        + b3_ref[...]


def _row_block(i):
    return (i, 0)


def _t1(y, dinvB, W):
    return pl.pallas_call(
        _t1_body,
        grid=(NBLK,),
        in_specs=[
            pl.BlockSpec((BLK, H), _row_block),
            pl.BlockSpec((BLK, H), _row_block),
            pl.BlockSpec((H, H), lambda i: (0, 0)),
        ],
        out_specs=pl.BlockSpec((BLK, H), _row_block),
        out_shape=jax.ShapeDtypeStruct((N, H), jnp.float32),
    )(y, dinvB, W)


def _t2(a0, a1, hs, dinvB, b2d, batch3):
    return pl.pallas_call(
        _t2_body,
        grid=(NBLK,),
        in_specs=[
            pl.BlockSpec((BLK, H), _row_block),
            pl.BlockSpec((BLK, H), _row_block),
            pl.BlockSpec((BLK, H), _row_block),
            pl.BlockSpec((BLK, H), _row_block),
            pl.BlockSpec((1, H), lambda i: (0, 0)),
            pl.BlockSpec((1, 1, BLK), lambda i: (i, 0, 0)),
        ],
        out_specs=[
            pl.BlockSpec((BLK, H), _row_block),
            pl.BlockSpec((G, H), lambda i: (0, 0)),
            pl.BlockSpec((G, H), lambda i: (0, 0)),
        ],
        out_shape=[
            jax.ShapeDtypeStruct((N, H), jnp.float32),
            jax.ShapeDtypeStruct((G, H), jnp.float32),
            jax.ShapeDtypeStruct((G, H), jnp.float32),
        ],
    )(a0, a1, hs, dinvB, b2d, batch3)


def kernel(x, edge_index, batch, W1, b1, W2, b2, W3, b3,
           L1W, L1b, L2W, L2b, L3W, L3b):
    i32 = jnp.int32
    pad = jnp.full((E_PAD - E,), N, i32)
    srcp = jnp.concatenate([edge_index[0].astype(i32), pad])
    dstp = jnp.concatenate([edge_index[1].astype(i32), pad])
    src2 = srcp.reshape(NW * CH, EC)
    dst2 = dstp.reshape(NW * CH, EC)

    zeros_in = jnp.zeros((N_PAD // NS, DW), jnp.float32)
    ones_in = jnp.ones((EC, DW), jnp.float32)
    deg_part = _deg_call(dst2, zeros_in, ones_in)
    d0 = deg_part[0, :, 0].reshape(N_PAD // 128, 128)
    d1 = deg_part[1, :, 0].reshape(N_PAD // 128, 128)
    dinv2d = pl.pallas_call(
        _prep_body,
        out_shape=jax.ShapeDtypeStruct((N_PAD // 128, 128), jnp.float32),
    )(d0, d1)
    dinv = dinv2d.reshape(N_PAD)[:N]
    dinvB = jnp.broadcast_to(dinv[:, None], (N, H))

    batch3 = batch.astype(i32).reshape(NBLK, 1, BLK)

    zerosH = jnp.zeros((ROWS_PER_TILE, H), jnp.float32)

    def layer(y_in, W, b):
        hs = _t1(y_in, dinvB, W)
        hs_pad = jnp.pad(hs, ((0, N_PAD - N), (0, 0)))
        acc = _agg_call(hs_pad, src2, dst2, zerosH)
        return _t2(acc[0, :N], acc[1, :N], hs, dinvB,
                   b.reshape(1, H), batch3)

    y1, p1, cnt = layer(x, W1, b1)
    y2, p2, _ = layer(y1, W2, b2)
    _, p3, _ = layer(y2, W3, b3)

    w2p = jnp.pad(L2W, ((0, 0), (0, H - L2W.shape[1])))
    b2p = jnp.pad(L2b, (0, H - L2b.shape[0])).reshape(1, H)
    w3p = jnp.pad(L3W, ((0, H - L3W.shape[0]), (0, H - L3W.shape[1])))
    b3p = jnp.pad(L3b, (0, H - L3b.shape[0])).reshape(1, H)

    out = pl.pallas_call(
        _head_body,
        out_shape=jax.ShapeDtypeStruct((G, H), jnp.float32),
    )(p1, p2, p3, cnt, L1W, L1b.reshape(1, H), w2p, b2p, w3p, b3p)
    return out[:, :L3W.shape[1]]


# speedup vs baseline: 1.0857x; 1.0857x over previous
import functools

import jax
import jax.numpy as jnp
from jax import lax
from jax.experimental import pallas as pl
from jax.experimental.pallas import tpu as pltpu
from jax.experimental.pallas import tpu_sc as plsc

N = 10000
E = 320000
G = 16
H = 128

NC = 2
NS = 16
NW = NC * NS
N_PAD = 10240
EC = 128
CH = 80
SEG = 40
CH0 = 120
CH1 = 2 * CH - CH0
E_PAD = NW * CH * EC
EPT = CH * EC
ROWS_PER_TILE = N_PAD // NS

_mesh = plsc.VectorSubcoreMesh(core_axis_name="c", subcore_axis_name="s")


DW = H


def _deg_body(dst_hbm, zeros_hbm, ones_hbm, deg_out, dst_v, ones_v, deg_sh):
    c = lax.axis_index("c")
    s = lax.axis_index("s")
    w = c * NS + s
    seg = N_PAD // NS

    pltpu.sync_copy(ones_hbm, ones_v)
    pltpu.sync_copy(zeros_hbm, deg_sh.at[pl.ds(s * seg, seg)])
    plsc.subcore_barrier()

    pltpu.sync_copy(dst_hbm.at[pl.ds(w * CH, CH)], dst_v)

    def _chunk(j, _):
        pltpu.sync_copy(ones_v, deg_sh.at[dst_v.at[j]], add=True)
        return 0
    lax.fori_loop(0, CH, _chunk, 0)

    plsc.subcore_barrier()
    sl = pl.ds(s * seg, seg)
    pltpu.sync_copy(deg_sh.at[sl], deg_out.at[c, sl])


_deg_call = pl.kernel(
    _deg_body,
    out_type=jax.ShapeDtypeStruct((NC, N_PAD, DW), jnp.float32),
    mesh=_mesh,
    scratch_types=[
        pltpu.VMEM((CH, EC), jnp.int32),
        pltpu.VMEM((EC, DW), jnp.float32),
        pltpu.VMEM_SHARED((N_PAD, DW), jnp.float32),
    ],
)


def _agg_body(hs_hbm, src_hbm, dst_hbm, zeros_hbm, out_hbm,
              src_v, dst_v, rows_v, acc_sh, sem0, sem1):
    c = lax.axis_index("c")
    s = lax.axis_index("s")
    w = c * NS + s

    pltpu.sync_copy(zeros_hbm, acc_sh.at[pl.ds(s * ROWS_PER_TILE, ROWS_PER_TILE)])
    plsc.subcore_barrier()

    def _core_loop(ch_c, base0):
        def _seg(g, _):
            base = base0 + g * SEG
            pltpu.sync_copy(src_hbm.at[pl.ds(base, SEG)], src_v)
            pltpu.sync_copy(dst_hbm.at[pl.ds(base, SEG)], dst_v)
            pltpu.async_copy(hs_hbm.at[src_v.at[0]], rows_v.at[0], sem0)

            def _pair(p, _):
                pltpu.async_copy(hs_hbm.at[src_v.at[2 * p + 1]], rows_v.at[1], sem1)
                pltpu.make_async_copy(hs_hbm.at[src_v.at[0]], rows_v.at[0], sem0).wait()
                pltpu.sync_copy(rows_v.at[0], acc_sh.at[dst_v.at[2 * p]], add=True)

                @pl.when(p < SEG // 2 - 1)
                def _():
                    pltpu.async_copy(hs_hbm.at[src_v.at[2 * p + 2]], rows_v.at[0], sem0)
                pltpu.make_async_copy(hs_hbm.at[src_v.at[0]], rows_v.at[1], sem1).wait()
                pltpu.sync_copy(rows_v.at[1], acc_sh.at[dst_v.at[2 * p + 1]], add=True)
                return 0
            lax.fori_loop(0, SEG // 2, _pair, 0)
            return 0
        lax.fori_loop(0, ch_c // SEG, _seg, 0)

    @pl.when(c == 0)
    def _():
        _core_loop(CH0, s * CH0)

    @pl.when(c == 1)
    def _():
        _core_loop(CH1, NS * CH0 + s * CH1)

    plsc.subcore_barrier()
    sl = pl.ds(s * ROWS_PER_TILE, ROWS_PER_TILE)
    pltpu.sync_copy(acc_sh.at[sl], out_hbm.at[c, sl])


_agg_call = pl.kernel(
    _agg_body,
    out_type=jax.ShapeDtypeStruct((NC, N_PAD, H), jnp.float32),
    mesh=_mesh,
    scratch_types=[
        pltpu.VMEM((SEG, EC), jnp.int32),
        pltpu.VMEM((SEG, EC), jnp.int32),
        pltpu.VMEM((2, EC, H), jnp.float32),
        pltpu.VMEM_SHARED((N_PAD, H), jnp.float32),
        pltpu.SemaphoreType.DMA,
        pltpu.SemaphoreType.DMA,
    ],
)


def _prep_body(d0_ref, d1_ref, dinv_ref):
    dinv_ref[...] = lax.rsqrt(d0_ref[...] + d1_ref[...] + 1.0)


def _t1_body(y_ref, d_ref, w_ref, o_ref):
    o_ref[...] = jnp.dot(y_ref[...] * d_ref[...], w_ref[...],
                         preferred_element_type=jnp.float32)


BLK = 2000
NBLK = N // BLK


def _t2_body(a0_ref, a1_ref, hs_ref, d_ref, b_ref, bt_ref,
             y_ref, pool_ref, cnt_ref):
    y = jnp.maximum(
        d_ref[...] * (a0_ref[...] + a1_ref[...] + hs_ref[...]) + b_ref[...],
        0.0)
    y_ref[...] = y
    bvec = bt_ref[0, 0, :]
    gi = lax.broadcasted_iota(jnp.int32, (G, BLK), 0)
    oh = (gi == bvec[None, :]).astype(jnp.float32)
    ps = jnp.dot(oh, y, preferred_element_type=jnp.float32)
    cs = jnp.broadcast_to(jnp.sum(oh, axis=1, keepdims=True), (G, H))

    @pl.when(pl.program_id(0) == 0)
    def _():
        pool_ref[...] = jnp.zeros_like(pool_ref)
        cnt_ref[...] = jnp.zeros_like(cnt_ref)

    pool_ref[...] += ps
    cnt_ref[...] += cs


def _head_body(s1_ref, s2_ref, s3_ref, cnt_ref,
               w1_ref, b1_ref, w2_ref, b2_ref, w3_ref, b3_ref, o_ref):
    z = (s1_ref[...] + s2_ref[...] + s3_ref[...]) / jnp.maximum(cnt_ref[...], 1.0)
    z = jnp.maximum(jnp.dot(z, w1_ref[...], preferred_element_type=jnp.float32)
                    + b1_ref[...], 0.0)
    z = jnp.maximum(jnp.dot(z, w2_ref[...], preferred_element_type=jnp.float32)
                    + b2_ref[...], 0.0)
    o_ref[...] = jnp.dot(z, w3_ref[...], preferred_element_type=jnp.float32) \
        + b3_ref[...]


def _row_block(i):
    return (i, 0)


def _t1(y, dinvB, W):
    return pl.pallas_call(
        _t1_body,
        grid=(NBLK,),
        in_specs=[
            pl.BlockSpec((BLK, H), _row_block),
            pl.BlockSpec((BLK, H), _row_block),
            pl.BlockSpec((H, H), lambda i: (0, 0)),
        ],
        out_specs=pl.BlockSpec((BLK, H), _row_block),
        out_shape=jax.ShapeDtypeStruct((N, H), jnp.float32),
    )(y, dinvB, W)


def _t2(a0, a1, hs, dinvB, b2d, batch3):
    return pl.pallas_call(
        _t2_body,
        grid=(NBLK,),
        in_specs=[
            pl.BlockSpec((BLK, H), _row_block),
            pl.BlockSpec((BLK, H), _row_block),
            pl.BlockSpec((BLK, H), _row_block),
            pl.BlockSpec((BLK, H), _row_block),
            pl.BlockSpec((1, H), lambda i: (0, 0)),
            pl.BlockSpec((1, 1, BLK), lambda i: (i, 0, 0)),
        ],
        out_specs=[
            pl.BlockSpec((BLK, H), _row_block),
            pl.BlockSpec((G, H), lambda i: (0, 0)),
            pl.BlockSpec((G, H), lambda i: (0, 0)),
        ],
        out_shape=[
            jax.ShapeDtypeStruct((N, H), jnp.float32),
            jax.ShapeDtypeStruct((G, H), jnp.float32),
            jax.ShapeDtypeStruct((G, H), jnp.float32),
        ],
    )(a0, a1, hs, dinvB, b2d, batch3)


def kernel(x, edge_index, batch, W1, b1, W2, b2, W3, b3,
           L1W, L1b, L2W, L2b, L3W, L3b):
    i32 = jnp.int32
    pad = jnp.full((E_PAD - E,), N, i32)
    srcp = jnp.concatenate([edge_index[0].astype(i32), pad])
    dstp = jnp.concatenate([edge_index[1].astype(i32), pad])
    src2 = srcp.reshape(NW * CH, EC)
    dst2 = dstp.reshape(NW * CH, EC)

    zeros_in = jnp.zeros((N_PAD // NS, DW), jnp.float32)
    ones_in = jnp.ones((EC, DW), jnp.float32)
    deg_part = _deg_call(dst2, zeros_in, ones_in)
    d0 = deg_part[0, :, 0].reshape(N_PAD // 128, 128)
    d1 = deg_part[1, :, 0].reshape(N_PAD // 128, 128)
    dinv2d = pl.pallas_call(
        _prep_body,
        out_shape=jax.ShapeDtypeStruct((N_PAD // 128, 128), jnp.float32),
    )(d0, d1)
    dinv = dinv2d.reshape(N_PAD)[:N]
    dinvB = jnp.broadcast_to(dinv[:, None], (N, H))

    batch3 = batch.astype(i32).reshape(NBLK, 1, BLK)

    zerosH = jnp.zeros((ROWS_PER_TILE, H), jnp.float32)

    def layer(y_in, W, b):
        hs = _t1(y_in, dinvB, W)
        hs_pad = jnp.pad(hs, ((0, N_PAD - N), (0, 0)))
        acc = _agg_call(hs_pad, src2, dst2, zerosH)
        return _t2(acc[0, :N], acc[1, :N], hs, dinvB,
                   b.reshape(1, H), batch3)

    y1, p1, cnt = layer(x, W1, b1)
    y2, p2, _ = layer(y1, W2, b2)
    _, p3, _ = layer(y2, W3, b3)

    w2p = jnp.pad(L2W, ((0, 0), (0, H - L2W.shape[1])))
    b2p = jnp.pad(L2b, (0, H - L2b.shape[0])).reshape(1, H)
    w3p = jnp.pad(L3W, ((0, H - L3W.shape[0]), (0, H - L3W.shape[1])))
    b3p = jnp.pad(L3b, (0, H - L3b.shape[0])).reshape(1, H)

    out = pl.pallas_call(
        _head_body,
        out_shape=jax.ShapeDtypeStruct((G, H), jnp.float32),
    )(p1, p2, p3, cnt, L1W, L1b.reshape(1, H), w2p, b2p, w3p, b3p)
    return out[:, :L3W.shape[1]]
